# Initial kernel scaffold; baseline (speedup 1.0000x reference)
import jax
import jax.numpy as jnp
from jax.experimental import pallas as pl


def kernel(x, projection_weights):
    raise NotImplementedError("write your pallas kernel here")


# speedup vs baseline: 16.1080x; 16.1080x over previous
import functools

import jax
import jax.numpy as jnp
from jax.experimental import pallas as pl
from jax.experimental.pallas import tpu as pltpu

_BISECT_ITERS = 20


def _threshold(p, kf):
    row_max = jnp.max(p, axis=1, keepdims=True)
    hi = jnp.maximum(row_max, 0.0) * (1.0 + 1e-6) + 1e-30
    lo = jnp.zeros_like(hi)

    def body(_, carry):
        lo, hi = carry
        mid = 0.5 * (lo + hi)
        cnt = jnp.sum((p >= mid).astype(jnp.float32), axis=1, keepdims=True)
        ge = cnt >= kf
        return jnp.where(ge, mid, lo), jnp.where(ge, hi, mid)

    lo, hi = jax.lax.fori_loop(0, _BISECT_ITERS, body, (lo, hi))
    return lo


def _fused_kernel(min_ref, x_ref, w_ref, o_ref, *, k):
    xb = x_ref[...]
    xb = jnp.where(min_ref[0, 0] >= 0.0, xb * 2.0 - 1.0, xb)
    p = jax.lax.dot_general(
        xb, w_ref[...],
        dimension_numbers=(((1,), (1,)), ((), ())),
        preferred_element_type=jnp.float32,
    )

    kf = jnp.float32(k)
    half = p.shape[0] // 2
    pa, pb = p[:half], p[half:]

    def body(_, carry):
        loa, hia, lob, hib = carry
        mida = 0.5 * (loa + hia)
        midb = 0.5 * (lob + hib)
        ca = jnp.sum((pa >= mida).astype(jnp.float32), axis=1, keepdims=True)
        cb = jnp.sum((pb >= midb).astype(jnp.float32), axis=1, keepdims=True)
        gea = ca >= kf
        geb = cb >= kf
        return (jnp.where(gea, mida, loa), jnp.where(gea, hia, mida),
                jnp.where(geb, midb, lob), jnp.where(geb, hib, midb))

    rma = jnp.max(pa, axis=1, keepdims=True)
    rmb = jnp.max(pb, axis=1, keepdims=True)
    hia0 = jnp.maximum(rma, 0.0) * (1.0 + 1e-6) + 1e-30
    hib0 = jnp.maximum(rmb, 0.0) * (1.0 + 1e-6) + 1e-30
    loa, hia, lob, hib = jax.lax.fori_loop(
        0, _BISECT_ITERS, body,
        (jnp.zeros_like(hia0), hia0, jnp.zeros_like(hib0), hib0))

    thr = jnp.concatenate([loa, lob], axis=0)

    enc = jnp.where(p >= thr, p, 0.0)
    nrm = jnp.sqrt(jnp.sum(enc * enc, axis=1, keepdims=True))
    o_ref[...] = enc / jnp.maximum(nrm, 1e-12)


def kernel(x, projection_weights):
    n, d_in = x.shape
    d_out = projection_weights.shape[0]
    k = max(1, int(d_out * 0.1))

    min_val = jnp.min(x).reshape(1, 1)

    block_rows = 256
    grid = (n // block_rows,)

    return pl.pallas_call(
        functools.partial(_fused_kernel, k=k),
        grid=grid,
        in_specs=[
            pl.BlockSpec(memory_space=pltpu.SMEM),
            pl.BlockSpec((block_rows, d_in), lambda i: (i, 0)),
            pl.BlockSpec((d_out, d_in), lambda i: (0, 0)),
        ],
        out_specs=pl.BlockSpec((block_rows, d_out), lambda i: (i, 0)),
        out_shape=jax.ShapeDtypeStruct((n, d_out), jnp.float32),
        compiler_params=pltpu.CompilerParams(
            dimension_semantics=("arbitrary",),
        ),
    )(min_val, x, projection_weights)
